# baseline (speedup 1.0000x reference)
import functools

import jax
import jax.numpy as jnp
from jax import lax
from jax.experimental import pallas as pl
from jax.experimental.pallas import tpu as pltpu
from jax.experimental.pallas import tpu_sc as plsc

_B = 4096
_F = 26
_D = 16
_ROWS_PER_FIELD = 40000
_NC = 2
_NS = 16
_NW = _NC * _NS
_BW = _B // _NW
_L = 16
_TBLK = (_ROWS_PER_FIELD * _F * _D) // 128


def _body(x_hbm, table_hbm, w_hbm, b_hbm, out_hbm,
          xv, blkv, colv, rowsA, rowsB, wv, bv, accv,
          semA, semB):
    wid = lax.axis_index("s") * _NC + lax.axis_index("c")
    base = wid * _BW

    pltpu.sync_copy(x_hbm.at[pl.ds(base, _BW), :], xv)
    pltpu.sync_copy(w_hbm, wv)
    pltpu.sync_copy(b_hbm, bv)

    iot = lax.iota(jnp.int32, _L)
    for f in range(_F):
        colf = jnp.full((_L,), f, jnp.int32)
        for c in range(_BW // _L):
            r = plsc.load_gather(xv, [iot + c * _L, colf]) + (f * _ROWS_PER_FIELD)
            blkv[f, pl.ds(c * _L, _L)] = lax.shift_right_logical(r, 3)
            colv[f, pl.ds(c * _L, _L)] = lax.shift_left(
                lax.bitwise_and(r, 7), 4)

    for g in range(_BW // _L):
        accv[pl.ds(g * _L, _L)] = bv[pl.ds(g * _L, _L)]

    def _fire(f, rows_ref, sem):
        return pltpu.async_copy(table_hbm.at[blkv.at[f]], rows_ref, sem)

    def _accum(f, rows_ref):
        for g in range(_BW // _L):
            rowvec = iot + g * _L
            colc = colv[f, pl.ds(g * _L, _L)]
            acc = accv[pl.ds(g * _L, _L)]
            for d in range(_D):
                vals = plsc.load_gather(rowsA if rows_ref is None else rows_ref,
                                        [rowvec, colc + d])
                acc = acc + vals * wv[f, pl.ds(d * _L, _L)]
            accv[pl.ds(g * _L, _L)] = acc

    cA = _fire(0, rowsA, semA)
    cB = _fire(1, rowsB, semB)
    del cA, cB

    def step(g, carry):
        f = g * 2
        pltpu.make_async_copy(table_hbm.at[blkv.at[f]], rowsA, semA).wait()

        @pl.when(f + 2 < _F)
        def _():
            _fire(f + 2, rowsA, semA)

        _accum(f, rowsA)
        pltpu.make_async_copy(table_hbm.at[blkv.at[f + 1]], rowsB, semB).wait()

        @pl.when(f + 3 < _F)
        def _():
            _fire(f + 3, rowsB, semB)

        _accum(f + 1, rowsB)
        return carry

    lax.fori_loop(0, _F // 2, step, 0)

    pltpu.sync_copy(accv, out_hbm.at[pl.ds(base, _BW)])


_sc_call = functools.partial(
    pl.kernel,
    out_type=jax.ShapeDtypeStruct((_B,), jnp.float32),
    mesh=plsc.VectorSubcoreMesh(core_axis_name="c", subcore_axis_name="s"),
    compiler_params=pltpu.CompilerParams(
        needs_layout_passes=False, use_tc_tiling_on_sc=True),
    scratch_types=[
        pltpu.VMEM((_BW, _F), jnp.int32),
        pltpu.VMEM((_F, _BW), jnp.int32),
        pltpu.VMEM((_F, _BW), jnp.int32),
        pltpu.VMEM((_BW, 128), jnp.float32),
        pltpu.VMEM((_BW, 128), jnp.float32),
        pltpu.VMEM((_F, _D * _L), jnp.float32),
        pltpu.VMEM((_BW,), jnp.float32),
        pltpu.VMEM((_BW,), jnp.float32),
        pltpu.SemaphoreType.DMA,
        pltpu.SemaphoreType.DMA,
    ],
)(_body)


def kernel(x, table, W, b, current_epoch, current_step):
    table2 = table.reshape(_TBLK, 128)
    wrep = jnp.repeat(W.reshape(_F, _D), _L, axis=1)
    b128 = jnp.broadcast_to(b.astype(jnp.float32), (_BW,))
    out = _sc_call(x, table2, wrep, b128)
    return out.reshape(_B, 1)


# speedup vs baseline: 2.7684x; 2.7684x over previous
import functools

import jax
import jax.numpy as jnp
from jax import lax
from jax.experimental import pallas as pl
from jax.experimental.pallas import tpu as pltpu
from jax.experimental.pallas import tpu_sc as plsc

_B = 4096
_F = 26
_D = 16
_RPF = 40000
_NC = 2
_NS = 16
_BH = _B // _NC
_L = 16
_C = 5888
_NCHUNK = 7


def _body(xt_hbm, table_hbm, w_hbm, b_hbm, out_hbm, slabs_hbm,
          xv, wvf, bufA, bufB, partial, red, binit, outv):
    c = lax.axis_index("c")
    s = lax.axis_index("s")
    base = c * _BH

    zero16 = jnp.zeros((_L,), jnp.float32)
    for q in range(_BH // _L):
        partial[q] = zero16

    def run_field(f):
        pltpu.sync_copy(
            xt_hbm.at[pl.ds(pl.multiple_of(f * _B + base, 128), _BH)], xv)
        pltpu.sync_copy(
            w_hbm.at[pl.ds(pl.multiple_of(f * (_D * _L), 128), _D * _L)], wvf)
        roff = f * _RPF
        l0 = roff - 64 * lax.bitwise_and(f, 1)

        for chunk in range(_NCHUNK):
            lc = pl.multiple_of(l0 + chunk * _C, 128)
            pltpu.sync_copy(table_hbm.at[pl.ds(0, 8), pl.ds(lc, _C)], bufA)
            pltpu.sync_copy(table_hbm.at[pl.ds(8, 8), pl.ds(lc, _C)], bufB)

            def group(g, carry):
                col = xv[pl.ds(g * _L, _L)] + (roff - lc)
                m = (col >= 0) & (col < _C)
                colc = jnp.clip(col, 0, _C - 1)
                acc = partial[g]
                for d in range(_D):
                    buf = bufA if d < 8 else bufB
                    svec = jnp.full((_L,), d % 8, jnp.int32)
                    val = plsc.load_gather(buf, [svec, colc])
                    val = jnp.where(m, val, jnp.float32(0.0))
                    acc = acc + val * wvf[pl.ds(d * _L, _L)]
                partial[g] = acc
                return carry

            lax.fori_loop(0, _BH // _L, group, 0)

    run_field(s)

    @pl.when(s + _NS < _F)
    def _():
        run_field(s + _NS)

    pltpu.sync_copy(partial, slabs_hbm.at[c * _NS + s])
    plsc.subcore_barrier()
    for t in range(_NS):
        pltpu.sync_copy(slabs_hbm.at[c * _NS + t, pl.ds(s * 8, 8), :],
                        red.at[t])
    pltpu.sync_copy(
        b_hbm.at[pl.ds(pl.multiple_of(base + s * 128, 128), 128)], binit)
    for r in range(8):
        acc = binit[pl.ds(r * _L, _L)]
        for t in range(_NS):
            acc = acc + red[t, r]
        outv[pl.ds(r * _L, _L)] = acc
    pltpu.sync_copy(
        outv, out_hbm.at[pl.ds(pl.multiple_of(base + s * 128, 128), 128)])


_sc_call = functools.partial(
    pl.kernel,
    out_type=(
        jax.ShapeDtypeStruct((_B,), jnp.float32),
        jax.ShapeDtypeStruct((_NC * _NS, _BH // _L, _L), jnp.float32),
    ),
    mesh=plsc.VectorSubcoreMesh(core_axis_name="c", subcore_axis_name="s"),
    compiler_params=pltpu.CompilerParams(
        needs_layout_passes=False, use_tc_tiling_on_sc=True),
    scratch_types=[
        pltpu.VMEM((_BH,), jnp.int32),
        pltpu.VMEM((_D * _L,), jnp.float32),
        pltpu.VMEM((8, _C), jnp.float32),
        pltpu.VMEM((8, _C), jnp.float32),
        pltpu.VMEM((_BH // _L, _L), jnp.float32),
        pltpu.VMEM((_NS, 8, _L), jnp.float32),
        pltpu.VMEM((128,), jnp.float32),
        pltpu.VMEM((128,), jnp.float32),
    ],
)(_body)


def kernel(x, table, W, b, current_epoch, current_step):
    xt = x.T.reshape(_F * _B)
    tablet = table.T
    wrep = jnp.repeat(W.reshape(_F, _D), _L, axis=1).reshape(_F * _D * _L)
    bfull = jnp.broadcast_to(b.astype(jnp.float32), (_B,))
    out, _unused_slabs = _sc_call(xt, tablet, wrep, bfull)
    return out.reshape(_B, 1)


# speedup vs baseline: 3.1490x; 1.1375x over previous
import functools

import jax
import jax.numpy as jnp
from jax import lax
from jax.experimental import pallas as pl
from jax.experimental.pallas import tpu as pltpu
from jax.experimental.pallas import tpu_sc as plsc

_B = 4096
_F = 26
_D = 16
_RPF = 40000
_NC = 2
_NS = 16
_BH = _B // _NC
_L = 16
_C = 5888
_NCHUNK = 7


def _body(xt_hbm, table_hbm, w_hbm, b_hbm, out_hbm, slabs_hbm,
          xv, wvf, bufA, bufB, partial, red, binit, outv):
    c = lax.axis_index("c")
    s = lax.axis_index("s")
    base = c * _BH

    zero16 = jnp.zeros((_L,), jnp.float32)
    for q in range(_BH // _L):
        partial[q] = zero16

    def run_field(f):
        pltpu.sync_copy(
            xt_hbm.at[pl.ds(pl.multiple_of(f * _B + base, 128), _BH)], xv)
        pltpu.sync_copy(
            w_hbm.at[pl.ds(pl.multiple_of(f * (_D * _L), 128), _D * _L)], wvf)
        roff = f * _RPF
        l0 = roff - 64 * lax.bitwise_and(f, 1)

        for chunk in range(_NCHUNK):
            lc = pl.multiple_of(l0 + chunk * _C, 128)
            pltpu.sync_copy(table_hbm.at[pl.ds(0, 8), pl.ds(lc, _C)], bufA)
            pltpu.sync_copy(table_hbm.at[pl.ds(8, 8), pl.ds(lc, _C)], bufB)

            def group(g, carry):
                col = xv[pl.ds(g * _L, _L)] + (roff - lc)
                m = (col >= 0) & (col < _C)
                colc = jnp.clip(col, 0, _C - 1)
                accs = [partial[g]] + [jnp.zeros((_L,), jnp.float32)] * 3
                for d in range(_D):
                    buf = bufA if d < 8 else bufB
                    svec = jnp.full((_L,), d % 8, jnp.int32)
                    val = plsc.load_gather(buf, [svec, colc], mask=m)
                    val = jnp.where(m, val, jnp.float32(0.0))
                    accs[d % 4] = accs[d % 4] + val * wvf[pl.ds(d * _L, _L)]
                partial[g] = (accs[0] + accs[1]) + (accs[2] + accs[3])
                return carry

            lax.fori_loop(0, _BH // _L, group, 0)

    run_field(s)

    @pl.when(s + _NS < _F)
    def _():
        run_field(s + _NS)

    pltpu.sync_copy(partial, slabs_hbm.at[c * _NS + s])
    plsc.subcore_barrier()
    for t in range(_NS):
        pltpu.sync_copy(slabs_hbm.at[c * _NS + t, pl.ds(s * 8, 8), :],
                        red.at[t])
    pltpu.sync_copy(
        b_hbm.at[pl.ds(pl.multiple_of(base + s * 128, 128), 128)], binit)
    for r in range(8):
        acc = binit[pl.ds(r * _L, _L)]
        for t in range(_NS):
            acc = acc + red[t, r]
        outv[pl.ds(r * _L, _L)] = acc
    pltpu.sync_copy(
        outv, out_hbm.at[pl.ds(pl.multiple_of(base + s * 128, 128), 128)])


_sc_call = functools.partial(
    pl.kernel,
    out_type=(
        jax.ShapeDtypeStruct((_B,), jnp.float32),
        jax.ShapeDtypeStruct((_NC * _NS, _BH // _L, _L), jnp.float32),
    ),
    mesh=plsc.VectorSubcoreMesh(core_axis_name="c", subcore_axis_name="s"),
    compiler_params=pltpu.CompilerParams(
        needs_layout_passes=False, use_tc_tiling_on_sc=True),
    scratch_types=[
        pltpu.VMEM((_BH,), jnp.int32),
        pltpu.VMEM((_D * _L,), jnp.float32),
        pltpu.VMEM((8, _C), jnp.float32),
        pltpu.VMEM((8, _C), jnp.float32),
        pltpu.VMEM((_BH // _L, _L), jnp.float32),
        pltpu.VMEM((_NS, 8, _L), jnp.float32),
        pltpu.VMEM((128,), jnp.float32),
        pltpu.VMEM((128,), jnp.float32),
    ],
)(_body)


def kernel(x, table, W, b, current_epoch, current_step):
    xt = x.T.reshape(_F * _B)
    tablet = table.T
    wrep = jnp.repeat(W.reshape(_F, _D), _L, axis=1).reshape(_F * _D * _L)
    bfull = jnp.broadcast_to(b.astype(jnp.float32), (_B,))
    out, _unused_slabs = _sc_call(xt, tablet, wrep, bfull)
    return out.reshape(_B, 1)
